# Initial kernel scaffold; baseline (speedup 1.0000x reference)
import jax
import jax.numpy as jnp
from jax.experimental import pallas as pl


def kernel(pos_x, neg_x, emb_table, pair_w, c):
    raise NotImplementedError("write your pallas kernel here")


# speedup vs baseline: 17.3695x; 17.3695x over previous
import functools

import jax
import jax.numpy as jnp
from jax import lax
from jax.experimental import pallas as pl
from jax.experimental.pallas import tpu as pltpu
from jax.experimental.pallas import tpu_sc as plsc

_NUM_DOMAINS = 5
_EMB_DIM = 16
_LANES = 16

_NUM_WORKERS = 32
_CHUNK = 512
_IDX_SUB = 128


def _make_sc_scorer(total_samples):
    per_w = total_samples // _NUM_WORKERS
    chunks = per_w // _CHUNK
    idx_per_chunk = _CHUNK * _NUM_DOMAINS
    n_sub = idx_per_chunk // _IDX_SUB

    info = plsc.get_sparse_core_info()
    nc = info.num_cores

    mesh = plsc.VectorSubcoreMesh(core_axis_name="c", subcore_axis_name="s")

    @functools.partial(
        pl.kernel,
        mesh=mesh,
        compiler_params=pltpu.CompilerParams(
            needs_layout_passes=False, use_tc_tiling_on_sc=False),
        out_type=jax.ShapeDtypeStruct((total_samples,), jnp.float32),
        scratch_types=[
            pltpu.VMEM((idx_per_chunk,), jnp.int32),
            pltpu.VMEM((idx_per_chunk,), jnp.int32),
            pltpu.VMEM((idx_per_chunk, _EMB_DIM), jnp.float32),
            pltpu.VMEM((_CHUNK,), jnp.float32),
            pltpu.VMEM((_LANES,), jnp.float32),
            pltpu.SemaphoreType.DMA,
        ],
    )
    def scorer(table_hbm, idx_hbm, par_hbm, out_hbm,
               idx_v, idxa_v, rows_v, out_v, par_v, sem):
        wid = lax.axis_index("s") * nc + lax.axis_index("c")
        base = wid * per_w

        pltpu.sync_copy(par_hbm, par_v)
        pv = jnp.exp(par_v[:])
        wh = pv[0] * 0.5
        ec = pv[1]
        lanes = lax.iota(jnp.int32, _LANES)

        def chunk_body(g, carry):
            cbase = base + g * _CHUNK
            pltpu.sync_copy(idx_hbm.at[pl.ds(cbase * _NUM_DOMAINS,
                                             idx_per_chunk)], idx_v)

            def mk_body(t, c1):
                v = idx_v[pl.ds(t * _LANES, _LANES)]
                idxa_v[pl.ds(t * _LANES, _LANES)] = v * 8
                return c1

            lax.fori_loop(0, idx_per_chunk // _LANES, mk_body, 0)

            copies = [
                pltpu.async_copy(
                    table_hbm.at[idxa_v.at[pl.ds(k * _IDX_SUB, _IDX_SUB)]],
                    rows_v.at[pl.ds(k * _IDX_SUB, _IDX_SUB)],
                    sem,
                )
                for k in range(n_sub)
            ]
            for cp in copies:
                cp.wait()

            def group_body(j, c2):
                gbase = j * (_LANES * _NUM_DOMAINS)
                vals = jnp.zeros((_LANES,), jnp.float32)
                for i in range(_LANES):
                    s5 = gbase + i * _NUM_DOMAINS
                    e0 = rows_v[s5, :]
                    e1 = rows_v[s5 + 1, :]
                    e2 = rows_v[s5 + 2, :]
                    e3 = rows_v[s5 + 3, :]
                    e4 = rows_v[s5 + 4, :]
                    sv = e0 + e1 + e2 + e3 + e4
                    q = (sv * sv - e0 * e0 - e1 * e1 - e2 * e2
                         - e3 * e3 - e4 * e4)
                    vals = jnp.where(lanes == i, jnp.sum(q), vals)
                out_v[pl.ds(j * _LANES, _LANES)] = jnp.exp(vals * wh) * ec
                return c2

            lax.fori_loop(0, _CHUNK // _LANES, group_body, 0)

            pltpu.sync_copy(out_v, out_hbm.at[pl.ds(cbase, _CHUNK)])
            return carry

        lax.fori_loop(0, chunks, chunk_body, 0)

    return scorer


def kernel(pos_x, neg_x, emb_table, pair_w, c):
    B, N, D = neg_x.shape
    total = B * (1 + N)
    all_idx = jnp.concatenate(
        [pos_x.reshape(-1), neg_x.reshape(-1)]).astype(jnp.int32)
    params = jnp.zeros((_LANES,), jnp.float32)
    params = params.at[0].set(pair_w[0]).at[1].set(c)
    V, E = emb_table.shape
    table_pad = jnp.pad(emb_table, ((0, 0), (0, 128 - E)))
    table8 = table_pad.reshape(V * 8, E)
    scores = _make_sc_scorer(total)(table8, all_idx, params)
    pos_score = scores[:B]
    neg_score = scores[B:].reshape(B, N)
    return pos_score, neg_score
